# baseline (speedup 1.0000x reference)
import functools

import jax
import jax.numpy as jnp
from jax import lax
from jax.experimental import pallas as pl
from jax.experimental.pallas import tpu as pltpu
from jax.experimental.pallas import tpu_sc as plsc

B = 100000
D = 256
C = 160
NC = 2
NS = 16
NW = NC * NS
NUM_CHUNKS = B // C
T = -(-NUM_CHUNKS // NW)
NBUF = 3
LAST_CUT = NUM_CHUNKS - (T - 1) * NW


@functools.partial(
    pl.kernel,
    mesh=plsc.VectorSubcoreMesh(core_axis_name="c", subcore_axis_name="s"),
    out_type=jax.ShapeDtypeStruct((B, D), jnp.float32),
    scratch_types=(
        [pltpu.VMEM((C,), jnp.int32)] * NBUF
        + [pltpu.VMEM((C, D), jnp.float32)] * NBUF
        + [pltpu.SemaphoreType.DMA] * (3 * NBUF)
    ),
)
def _gather_kernel(idx_hbm, table_hbm, out_hbm, *scratch):
    idx_v = scratch[0:NBUF]
    rows_v = scratch[NBUF:2 * NBUF]
    sems = scratch[2 * NBUF:]
    isem = sems[0:NBUF]
    gsem = sems[NBUF:2 * NBUF]
    osem = sems[2 * NBUF:3 * NBUF]

    wid = lax.axis_index("s") * NC + lax.axis_index("c")

    def run(t, fn):
        if t == T - 1:
            pl.when(wid < LAST_CUT)(fn)
        else:
            fn()

    def start_idx(t):
        base = (wid + t * NW) * C
        pltpu.async_copy(idx_hbm.at[pl.ds(base, C)],
                         idx_v[t % NBUF], isem[t % NBUF])

    def wait_idx(t):
        pltpu.make_async_copy(idx_hbm.at[pl.ds(0, C)],
                              idx_v[t % NBUF], isem[t % NBUF]).wait()

    def start_gather(t):
        pltpu.async_copy(table_hbm.at[idx_v[t % NBUF]],
                         rows_v[t % NBUF], gsem[t % NBUF])

    def wait_gather(t):
        pltpu.make_async_copy(out_hbm.at[pl.ds(0, C)],
                              rows_v[t % NBUF], gsem[t % NBUF]).wait()

    def start_store(t):
        base = (wid + t * NW) * C
        pltpu.async_copy(rows_v[t % NBUF],
                         out_hbm.at[pl.ds(base, C)], osem[t % NBUF])

    def wait_store(t):
        pltpu.make_async_copy(rows_v[t % NBUF],
                              out_hbm.at[pl.ds(0, C)], osem[t % NBUF]).wait()

    run(0, lambda: start_idx(0))
    run(1, lambda: start_idx(1))

    for t in range(T):
        def stage_a(t=t):
            if t >= NBUF:
                wait_store(t - NBUF)
            wait_idx(t)
            start_gather(t)
        run(t, stage_a)

        if t >= 1:
            def stage_b(t=t):
                wait_gather(t - 1)
                start_store(t - 1)
            run(t - 1, stage_b)

        if t + 2 < T:
            run(t + 2, lambda t=t: start_idx(t + 2))

    def last_chunk(t=T - 1):
        wait_gather(t)
        start_store(t)
    run(T - 1, last_chunk)
    pl.when(wid >= LAST_CUT)(lambda: wait_store(T - 1 - NBUF))
    for t in range(T - NBUF, T - 1):
        wait_store(t)
    pl.when(wid < LAST_CUT)(lambda: wait_store(T - 1))


def kernel(atomic_numbers, table):
    idx = atomic_numbers.astype(jnp.int32)
    return _gather_kernel(idx, table)
